# Initial kernel scaffold; baseline (speedup 1.0000x reference)
import jax
import jax.numpy as jnp
from jax.experimental import pallas as pl


def kernel(input, lengths, span_indexes, W, b):
    raise NotImplementedError("write your pallas kernel here")


# speedup vs baseline: 10.1989x; 10.1989x over previous
import jax
import jax.numpy as jnp
from jax.experimental import pallas as pl
from jax.experimental.pallas import tpu as pltpu


def _agg_kernel(len_ref, starts_ref, ends_ref, x_ref, W_ref, b_ref, out_ref):
    _, L, _ = starts_ref.shape
    T = x_ref.shape[1]
    ii = starts_ref[0]
    jj = ends_ref[0]
    t = jax.lax.broadcasted_iota(jnp.int32, (L, T), 1)
    mask = (t >= ii) & (t < jj)
    width = (jj - ii).astype(jnp.float32)
    j_iota = jax.lax.broadcasted_iota(jnp.int32, (L, 1), 0)
    valid = (j_iota < len_ref[pl.program_id(0), 0]).astype(jnp.float32)
    M = jnp.where(mask, valid / width, 0.0)
    agg = jnp.dot(M, x_ref[0], preferred_element_type=jnp.float32)
    out_ref[0] = (
        jnp.dot(agg, W_ref[...].T, preferred_element_type=jnp.float32)
        + b_ref[...]
    )


def kernel(input, lengths, span_indexes, W, b):
    B, T, D = input.shape
    L = span_indexes.shape[1]
    starts = span_indexes[..., 0:1]
    ends = span_indexes[..., 1:2]
    b2 = b.reshape(1, D)

    out = pl.pallas_call(
        _agg_kernel,
        grid=(B,),
        in_specs=[
            pl.BlockSpec(lengths.shape + (1,), lambda i: (0, 0), memory_space=pltpu.SMEM),
            pl.BlockSpec((1, L, 1), lambda i: (i, 0, 0)),
            pl.BlockSpec((1, L, 1), lambda i: (i, 0, 0)),
            pl.BlockSpec((1, T, D), lambda i: (i, 0, 0)),
            pl.BlockSpec((D, D), lambda i: (0, 0)),
            pl.BlockSpec((1, D), lambda i: (0, 0)),
        ],
        out_specs=pl.BlockSpec((1, L, D), lambda i: (i, 0, 0)),
        out_shape=jax.ShapeDtypeStruct((B, L, D), jnp.float32),
        compiler_params=pltpu.CompilerParams(
            dimension_semantics=("arbitrary",),
        ),
    )(lengths.reshape(B, 1), starts, ends, input, W, b2)
    return out


# speedup vs baseline: 10.2103x; 1.0011x over previous
import jax
import jax.numpy as jnp
from jax.experimental import pallas as pl
from jax.experimental.pallas import tpu as pltpu


def _agg_kernel(len_ref, starts_ref, ends_ref, x_ref, W_ref, b_ref, out_ref):
    _, L, _ = starts_ref.shape
    T = x_ref.shape[1]
    ii = starts_ref[0]
    jj = ends_ref[0]
    t = jax.lax.broadcasted_iota(jnp.int32, (L, T), 1)
    mask = (t >= ii) & (t < jj)
    width = (jj - ii).astype(jnp.float32)
    j_iota = jax.lax.broadcasted_iota(jnp.int32, (L, 1), 0)
    valid = (j_iota < len_ref[pl.program_id(0), 0]).astype(jnp.float32)
    M = jnp.where(mask, valid / width, 0.0)
    agg = jnp.dot(
        M.astype(jnp.bfloat16),
        x_ref[0].astype(jnp.bfloat16),
        preferred_element_type=jnp.float32,
    )
    out_ref[0] = (
        jnp.dot(
            agg.astype(jnp.bfloat16),
            W_ref[...].T.astype(jnp.bfloat16),
            preferred_element_type=jnp.float32,
        )
        + b_ref[...]
    )


def kernel(input, lengths, span_indexes, W, b):
    B, T, D = input.shape
    L = span_indexes.shape[1]
    starts = span_indexes[..., 0:1]
    ends = span_indexes[..., 1:2]
    b2 = b.reshape(1, D)

    out = pl.pallas_call(
        _agg_kernel,
        grid=(B,),
        in_specs=[
            pl.BlockSpec(lengths.shape + (1,), lambda i: (0, 0), memory_space=pltpu.SMEM),
            pl.BlockSpec((1, L, 1), lambda i: (i, 0, 0)),
            pl.BlockSpec((1, L, 1), lambda i: (i, 0, 0)),
            pl.BlockSpec((1, T, D), lambda i: (i, 0, 0)),
            pl.BlockSpec((D, D), lambda i: (0, 0)),
            pl.BlockSpec((1, D), lambda i: (0, 0)),
        ],
        out_specs=pl.BlockSpec((1, L, D), lambda i: (i, 0, 0)),
        out_shape=jax.ShapeDtypeStruct((B, L, D), jnp.float32),
        compiler_params=pltpu.CompilerParams(
            dimension_semantics=("arbitrary",),
        ),
    )(lengths.reshape(B, 1), starts, ends, input, W, b2)
    return out


# speedup vs baseline: 10.2348x; 1.0024x over previous
import jax
import jax.numpy as jnp
from jax.experimental import pallas as pl
from jax.experimental.pallas import tpu as pltpu


def _agg_kernel(len_ref, starts_ref, ends_ref, x_ref, W_ref, b_ref, out_ref):
    _, L, _ = starts_ref.shape
    T = x_ref.shape[1]
    ii = starts_ref[0]
    jj = ends_ref[0]
    t = jax.lax.broadcasted_iota(jnp.int32, (L, T), 1)
    mask = (t >= ii) & (t < jj)
    width = (jj - ii).astype(jnp.float32)
    j_iota = jax.lax.broadcasted_iota(jnp.int32, (L, 1), 0)
    valid = (j_iota < len_ref[pl.program_id(0), 0]).astype(jnp.float32)
    M = jnp.where(mask, valid / width, 0.0)
    agg = jnp.dot(
        M.astype(jnp.bfloat16),
        x_ref[0].astype(jnp.bfloat16),
        preferred_element_type=jnp.float32,
    )
    out_ref[0] = (
        jnp.dot(
            agg.astype(jnp.bfloat16),
            W_ref[...].T.astype(jnp.bfloat16),
            preferred_element_type=jnp.float32,
        )
        + b_ref[...]
    )


def kernel(input, lengths, span_indexes, W, b):
    B, T, D = input.shape
    L = span_indexes.shape[1]
    starts = span_indexes[..., 0:1]
    ends = span_indexes[..., 1:2]
    b2 = b.reshape(1, D)

    out = pl.pallas_call(
        _agg_kernel,
        grid=(B,),
        in_specs=[
            pl.BlockSpec(lengths.shape + (1,), lambda i: (0, 0), memory_space=pltpu.SMEM),
            pl.BlockSpec((1, L, 1), lambda i: (i, 0, 0)),
            pl.BlockSpec((1, L, 1), lambda i: (i, 0, 0)),
            pl.BlockSpec((1, T, D), lambda i: (i, 0, 0)),
            pl.BlockSpec((D, D), lambda i: (0, 0)),
            pl.BlockSpec((1, D), lambda i: (0, 0)),
        ],
        out_specs=pl.BlockSpec((1, L, D), lambda i: (i, 0, 0)),
        out_shape=jax.ShapeDtypeStruct((B, L, D), jnp.float32),
        compiler_params=pltpu.CompilerParams(
            dimension_semantics=("parallel",),
        ),
    )(lengths.reshape(B, 1), starts, ends, input, W, b2)
    return out
